# baseline (speedup 1.0000x reference)
import functools

import jax
import jax.numpy as jnp
from jax import lax
from jax.experimental import pallas as pl
from jax.experimental.pallas import tpu as pltpu
from jax.experimental.pallas import tpu_sc as plsc

H = 1024
I = 256
ER = 15
EP = 16
T = 4096
TOPK2 = 2
TB = 512
CB = 256
NB = 48
PADT = NB * CB
SENT = TOPK2 * T
Y2R = SENT + CB
NS = 16
L = 16
CAP = T + 64
GC = 32
SC_C = 32
PC = 128
POSN = TOPK2 * T + 64
RPT = PADT // 32
HW = H // 2


def _silu(v):
    return v * jax.nn.sigmoid(v)


def _dotT(a, b):
    return lax.dot_general(a, b, (((1,), (1,)), ((), ())),
                           preferred_element_type=jnp.float32)


def _router_body(x_ref, wr_ref, rb_ref, e1_ref, e2_ref, w1_ref, w2_ref):
    x = x_ref[...]
    logits = _dotT(x, wr_ref[...]) + rb_ref[...]
    probs = jax.nn.sigmoid(logits)
    idx = lax.broadcasted_iota(jnp.int32, probs.shape, 1)
    v1 = jnp.max(probs, axis=1, keepdims=True)
    i1 = jnp.min(jnp.where(probs == v1, idx, ER), axis=1, keepdims=True)
    p2 = jnp.where(idx == i1, -jnp.inf, probs)
    v2 = jnp.max(p2, axis=1, keepdims=True)
    i2 = jnp.min(jnp.where(p2 == v2, idx, ER), axis=1, keepdims=True)
    den = v1 + v2
    e1_ref[...] = i1
    e2_ref[...] = i2
    w1_ref[...] = v1 / den
    w2_ref[...] = v2 / den


def _router(xs, Wr, rb):
    outs = pl.pallas_call(
        _router_body,
        grid=(T // TB,),
        in_specs=[
            pl.BlockSpec((TB, H), lambda i: (i, 0)),
            pl.BlockSpec((ER, H), lambda i: (0, 0)),
            pl.BlockSpec((ER,), lambda i: (0,)),
        ],
        out_specs=[
            pl.BlockSpec((TB, 1), lambda i: (i, 0)),
            pl.BlockSpec((TB, 1), lambda i: (i, 0)),
            pl.BlockSpec((TB, 1), lambda i: (i, 0)),
            pl.BlockSpec((TB, 1), lambda i: (i, 0)),
        ],
        out_shape=[
            jax.ShapeDtypeStruct((T, 1), jnp.int32),
            jax.ShapeDtypeStruct((T, 1), jnp.int32),
            jax.ShapeDtypeStruct((T, 1), jnp.float32),
            jax.ShapeDtypeStruct((T, 1), jnp.float32),
        ],
    )(xs, Wr, rb)
    return outs


def _dispatch_body(e1_hbm, e2_hbm, w1_hbm, w2_hbm, x_hbm,
                   xg_hbm, wg_hbm, pos_hbm, be_hbm,
                   e1_v, e2_v, w1_v, w2_v, stok_v,
                   tmp_v, allc_v, counts_sh, sa2p_v, vals_v, swb_v,
                   rows_v, rows2_v, sem, sem2):
    sid = lax.axis_index("s")
    de1 = pltpu.make_async_copy(e1_hbm, e1_v, sem)
    de2 = pltpu.make_async_copy(e2_hbm, e2_v, sem2)
    de1.start()
    de2.start()
    lanes = lax.broadcasted_iota(jnp.int32, (L,), 0)
    zero_vec = jnp.zeros((L,), jnp.int32)

    def _shift(v, k):
        sh = v.at[jnp.maximum(lanes - k, 0)].get(mode="promise_in_bounds")
        return v + jnp.where(lanes >= k, sh, 0)

    def _prefix(v):
        for k in (1, 2, 4, 8):
            v = _shift(v, k)
        return v

    def _splat(v, i):
        return v.at[zero_vec + i].get(mode="promise_in_bounds")

    def init_body(i, c):
        stok_v[pl.ds(i * L, L)] = zero_vec
        return c
    lax.fori_loop(0, CAP // L, init_body, 0)
    de1.wait()
    de2.wait()
    dw1 = pltpu.make_async_copy(w1_hbm, w1_v, sem)
    dw2 = pltpu.make_async_copy(w2_hbm, w2_v, sem2)
    dw1.start()
    dw2.start()

    def scan_pass(e_v):
        def body(s2, cv):
            o = s2 * (2 * L)
            ids_a = e_v[pl.ds(o, L)]
            ids_b = e_v[pl.ds(o + L, L)]
            m_a = ids_a == sid
            m_b = ids_b == sid
            pref_a = _prefix(jnp.where(m_a, 1, 0))
            pref_b = _prefix(jnp.where(m_b, 1, 0))
            tot_a = _splat(pref_a, L - 1)
            plsc.store_scatter(stok_v, [cv + pref_a - 1], o + lanes, mask=m_a)
            plsc.store_scatter(stok_v, [cv + tot_a + pref_b - 1],
                               o + L + lanes, mask=m_b)
            return cv + tot_a + _splat(pref_b, L - 1)
        return body

    cv1 = lax.fori_loop(0, T // (2 * L), scan_pass(e1_v), zero_vec)
    cv = lax.fori_loop(0, T // (2 * L), scan_pass(e2_v), cv1)

    dw1.wait()
    dw2.wait()

    tmp_v[...] = jnp.where(lanes == sid, cv, 0)
    pltpu.sync_copy(tmp_v, counts_sh.at[sid])
    plsc.subcore_barrier()
    pltpu.sync_copy(counts_sh, allc_v)
    counts = jnp.zeros((L,), jnp.int32)
    for _e in range(NS):
        counts = counts + allc_v[_e]
    padded = ((counts + (CB - 1)) >> 8) << 8
    pincl = _prefix(padded)
    basev = pincl - padded
    cursor = cv[0]
    c1 = cv1[0]
    base = _splat(basev, sid)[0]

    nch = (cursor + (GC - 1)) >> 5

    def _gd(c, buf, s):
        idx = stok_v.at[pl.ds(pl.multiple_of(c * GC, L), GC)]
        return pltpu.make_async_copy(x_hbm.at[idx], buf, s)

    def _write(c, buf):
        pltpu.sync_copy(
            buf, xg_hbm.at[pl.ds(pl.multiple_of(base + c * GC, GC), GC)])

    def gather_pair(p, _):
        c0 = p * 2
        d0 = _gd(c0, rows_v, sem)
        d1 = _gd(c0 + 1, rows2_v, sem2)
        d0.start()
        d1.start()
        d0.wait()

        @pl.when(c0 < nch)
        def _w0():
            _write(c0, rows_v)
        d1.wait()

        @pl.when(c0 + 1 < nch)
        def _w1():
            _write(c0 + 1, rows2_v)
        return 0
    lax.fori_loop(0, (nch + 1) >> 1, gather_pair, 0)

    def wg_chunk(j, _):
        for q in range(CB // L):
            r0 = pl.multiple_of(j * CB, CB) + q * L
            toks = stok_v[pl.ds(r0, L)]
            wa = plsc.load_gather(w1_v, [toks])
            wb = plsc.load_gather(w2_v, [toks])
            swb_v[pl.ds(q * L, L)] = jnp.where(r0 + lanes >= c1, wb, wa)
        pltpu.sync_copy(
            swb_v, wg_hbm.at[pl.ds(pl.multiple_of(base + j * CB, CB), CB)])
        return 0
    lax.fori_loop(0, (cursor + (CB - 1)) >> 8, wg_chunk, 0)

    def pos_chunk(c, _):
        for q in range(PC // L):
            r0 = pl.multiple_of(c * PC, PC) + q * L
            r = r0 + lanes
            toks = stok_v[pl.ds(r0, L)]
            slot = toks + jnp.where(r >= c1, T, 0)
            slot = jnp.where(r >= cursor, SENT, slot)
            sa2p_v.at[c][pl.ds(q * L, L)] = slot
            vals_v[pl.ds(q * L, L)] = base + r
        d = pltpu.make_async_copy(vals_v, pos_hbm.at[sa2p_v.at[c]], sem)
        d.start()
        d.wait()
        return 0
    lax.fori_loop(0, (cursor + (PC - 1)) >> 7, pos_chunk, 0)

    @pl.when(sid == EP - 1)
    def _tail():
        bblk = basev >> 8

        def be_slice(s, _):
            bv = s * L + lanes
            acc = jnp.zeros((L,), jnp.int32)
            for _e in range(EP):
                acc = acc + jnp.where(bv >= _splat(bblk, _e), 1, 0)
            tmp_v[...] = acc - 1
            pltpu.sync_copy(tmp_v, be_hbm.at[pl.ds(pl.multiple_of(s * L, L), L)])
            return 0
        lax.fori_loop(0, NB // L, be_slice, 0)


def _dispatch(e1, e2, w1, w2, xs):
    mesh = plsc.VectorSubcoreMesh(core_axis_name="c", subcore_axis_name="s",
                                  num_cores=1)
    f = pl.kernel(
        _dispatch_body,
        mesh=mesh,
        compiler_params=pltpu.CompilerParams(needs_layout_passes=False),
        out_type=[
            jax.ShapeDtypeStruct((PADT, H), jnp.float32),
            jax.ShapeDtypeStruct((PADT,), jnp.float32),
            jax.ShapeDtypeStruct((POSN,), jnp.int32),
            jax.ShapeDtypeStruct((NB,), jnp.int32),
        ],
        scratch_types=[
            pltpu.VMEM((T,), jnp.int32),
            pltpu.VMEM((T,), jnp.int32),
            pltpu.VMEM((T,), jnp.float32),
            pltpu.VMEM((T,), jnp.float32),
            pltpu.VMEM((CAP,), jnp.int32),
            pltpu.VMEM((L,), jnp.int32),
            pltpu.VMEM((NS, L), jnp.int32),
            pltpu.VMEM_SHARED((NS, L), jnp.int32),
            pltpu.VMEM((T // PC, PC), jnp.int32),
            pltpu.VMEM((PC,), jnp.int32),
            pltpu.VMEM((CB,), jnp.float32),
            pltpu.VMEM((GC, H), jnp.float32),
            pltpu.VMEM((GC, H), jnp.float32),
            pltpu.SemaphoreType.DMA,
            pltpu.SemaphoreType.DMA,
        ],
    )
    return f(e1, e2, w1, w2, xs)


def _group_body(be_ref, xg_ref, wg_ref, rg_ref, ru_ref, rd_ref, yg_ref):
    xb = xg_ref[...].astype(jnp.bfloat16)
    g = _dotT(xb, rg_ref[0])
    u = _dotT(xb, ru_ref[0])
    h = (_silu(g) * u * wg_ref[...]).astype(jnp.bfloat16)
    yg_ref[...] = _dotT(h, rd_ref[0])


def _grouped(be, xg, wg, rg_pad, ru_pad, rd_pad):
    grid_spec = pltpu.PrefetchScalarGridSpec(
        num_scalar_prefetch=1,
        grid=(NB,),
        in_specs=[
            pl.BlockSpec((CB, H), lambda j, be: (j, 0)),
            pl.BlockSpec((CB, 1), lambda j, be: (j, 0)),
            pl.BlockSpec((1, I, H), lambda j, be: (be[j], 0, 0)),
            pl.BlockSpec((1, I, H), lambda j, be: (be[j], 0, 0)),
            pl.BlockSpec((1, H, I), lambda j, be: (be[j], 0, 0)),
        ],
        out_specs=pl.BlockSpec((CB, H), lambda j, be: (j, 0)),
    )
    return pl.pallas_call(
        _group_body,
        grid_spec=grid_spec,
        out_shape=jax.ShapeDtypeStruct((PADT, H), jnp.float32),
    )(be, xg, wg.reshape(PADT, 1), rg_pad, ru_pad, rd_pad)


def _gcombine_body(yg_hbm, pos_hbm, y0_hbm, y1_hbm,
                   pos_v, rows_v, rows2_v, sem, sem2):
    wid = lax.axis_index("c") * NS + lax.axis_index("s")
    t0 = pl.multiple_of(wid * (T // 32), T // 32)
    pltpu.sync_copy(pos_hbm.at[pl.ds(t0, T // 32)], pos_v.at[0])
    pltpu.sync_copy(pos_hbm.at[pl.ds(T + t0, T // 32)], pos_v.at[1])
    nc = (T // 32) // SC_C

    def _gd(k, c, buf, s):
        idx = pos_v.at[k, pl.ds(c * SC_C, SC_C)]
        return pltpu.make_async_copy(yg_hbm.at[idx], buf, s)

    def _wr(dst, c, buf):
        pltpu.sync_copy(
            buf, dst.at[pl.ds(pl.multiple_of(t0 + c * SC_C, SC_C), SC_C)])

    for k, dst in ((0, y0_hbm), (1, y1_hbm)):
        for c in range(0, nc, 2):
            d0 = _gd(k, c, rows_v, sem)
            d1 = _gd(k, c + 1, rows2_v, sem2)
            d0.start()
            d1.start()
            d0.wait()
            _wr(dst, c, rows_v)
            d1.wait()
            _wr(dst, c + 1, rows2_v)


def _gcombine(yg, pos):
    mesh = plsc.VectorSubcoreMesh(core_axis_name="c", subcore_axis_name="s")
    f = pl.kernel(
        _gcombine_body,
        mesh=mesh,
        compiler_params=pltpu.CompilerParams(needs_layout_passes=False),
        out_type=[
            jax.ShapeDtypeStruct((T, H), jnp.float32),
            jax.ShapeDtypeStruct((T, H), jnp.float32),
        ],
        scratch_types=[
            pltpu.VMEM((2, T // 32), jnp.int32),
            pltpu.VMEM((SC_C, H), jnp.float32),
            pltpu.VMEM((SC_C, H), jnp.float32),
            pltpu.SemaphoreType.DMA,
            pltpu.SemaphoreType.DMA,
        ],
    )
    return f(yg, pos)


def _shared_body(x_ref, sg_ref, su_ref, sd_ref, out_ref):
    x = x_ref[...]
    g = _dotT(x, sg_ref[...])
    u = _dotT(x, su_ref[...])
    out_ref[...] = _dotT(_silu(g) * u, sd_ref[...])


def _shared(xs, sg, su, sd):
    return pl.pallas_call(
        _shared_body,
        grid=(T // TB,),
        in_specs=[
            pl.BlockSpec((TB, H), lambda i: (i, 0)),
            pl.BlockSpec((I, H), lambda i: (0, 0)),
            pl.BlockSpec((I, H), lambda i: (0, 0)),
            pl.BlockSpec((H, I), lambda i: (0, 0)),
        ],
        out_specs=pl.BlockSpec((TB, H), lambda i: (i, 0)),
        out_shape=jax.ShapeDtypeStruct((T, H), jnp.float32),
    )(xs, sg, su, sd)


def _combine_body(sh_ref, y0_ref, y1_ref, out_ref):
    out_ref[...] = sh_ref[...] + y0_ref[...] + y1_ref[...]


def _combine(sh, y0, y1):
    return pl.pallas_call(
        _combine_body,
        grid=(T // TB,),
        in_specs=[
            pl.BlockSpec((TB, H), lambda i: (i, 0)),
            pl.BlockSpec((TB, H), lambda i: (i, 0)),
            pl.BlockSpec((TB, H), lambda i: (i, 0)),
        ],
        out_specs=pl.BlockSpec((TB, H), lambda i: (i, 0)),
        out_shape=jax.ShapeDtypeStruct((T, H), jnp.float32),
    )(sh, y0, y1)


def kernel(x, sg, su, sd, rg, ru, rd, Wr, rb):
    orig_shape = x.shape
    xs = x.reshape(-1, H)
    e1, e2, w1, w2 = _router(xs, Wr, rb)
    xg, wg, pos, be = _dispatch(e1.reshape(-1), e2.reshape(-1),
                                w1.reshape(-1), w2.reshape(-1), xs)
    zpad = jnp.zeros((1,) + rg.shape[1:], jnp.bfloat16)
    rg_pad = jnp.concatenate([rg.astype(jnp.bfloat16), zpad], axis=0)
    ru_pad = jnp.concatenate([ru.astype(jnp.bfloat16), zpad], axis=0)
    rd_pad = jnp.concatenate(
        [rd.astype(jnp.bfloat16),
         jnp.zeros((1,) + rd.shape[1:], jnp.bfloat16)], axis=0)
    sh = _shared(xs, sg, su, sd)
    yg = _grouped(be, xg, wg, rg_pad, ru_pad, rd_pad)
    y0, y1 = _gcombine(yg, pos)
    out = _combine(sh, y0, y1)
    return out.reshape(orig_shape)


# speedup vs baseline: 1.5905x; 1.5905x over previous
import functools

import jax
import jax.numpy as jnp
from jax import lax
from jax.experimental import pallas as pl
from jax.experimental.pallas import tpu as pltpu
from jax.experimental.pallas import tpu_sc as plsc

H = 1024
I = 256
ER = 15
EP = 16
T = 4096
TOPK2 = 2
TB = 512
CB = 256
NB = 48
PADT = NB * CB
SENT = TOPK2 * T
Y2R = SENT + CB
NS = 16
L = 16
CAP = T + 64
GC = 32
SC_C = 32
PC = 128
POSN = TOPK2 * T + CB
RPT = PADT // 32
HW = H // 2


def _silu(v):
    return v * jax.nn.sigmoid(v)


def _dotT(a, b):
    return lax.dot_general(a, b, (((1,), (1,)), ((), ())),
                           preferred_element_type=jnp.float32)


def _router_body(x_ref, wr_ref, rb_ref, e1_ref, e2_ref, w1_ref, w2_ref):
    x = x_ref[...]
    logits = _dotT(x, wr_ref[...]) + rb_ref[...]
    probs = jax.nn.sigmoid(logits)
    idx = lax.broadcasted_iota(jnp.int32, probs.shape, 1)
    v1 = jnp.max(probs, axis=1, keepdims=True)
    i1 = jnp.min(jnp.where(probs == v1, idx, ER), axis=1, keepdims=True)
    p2 = jnp.where(idx == i1, -jnp.inf, probs)
    v2 = jnp.max(p2, axis=1, keepdims=True)
    i2 = jnp.min(jnp.where(p2 == v2, idx, ER), axis=1, keepdims=True)
    den = v1 + v2
    e1_ref[...] = i1
    e2_ref[...] = i2
    w1_ref[...] = v1 / den
    w2_ref[...] = v2 / den


def _router(xs, Wr, rb):
    outs = pl.pallas_call(
        _router_body,
        grid=(T // TB,),
        in_specs=[
            pl.BlockSpec((TB, H), lambda i: (i, 0)),
            pl.BlockSpec((ER, H), lambda i: (0, 0)),
            pl.BlockSpec((ER,), lambda i: (0,)),
        ],
        out_specs=[
            pl.BlockSpec((TB, 1), lambda i: (i, 0)),
            pl.BlockSpec((TB, 1), lambda i: (i, 0)),
            pl.BlockSpec((TB, 1), lambda i: (i, 0)),
            pl.BlockSpec((TB, 1), lambda i: (i, 0)),
        ],
        out_shape=[
            jax.ShapeDtypeStruct((T, 1), jnp.int32),
            jax.ShapeDtypeStruct((T, 1), jnp.int32),
            jax.ShapeDtypeStruct((T, 1), jnp.float32),
            jax.ShapeDtypeStruct((T, 1), jnp.float32),
        ],
    )(xs, Wr, rb)
    return outs


def _dispatch_body(e1_hbm, e2_hbm, w1_hbm, w2_hbm, x_hbm,
                   xg_hbm, wg_hbm, pos_hbm, be_hbm,
                   e1_v, e2_v, w1_v, w2_v, stok_v,
                   tmp_v, allc_v, counts_sh, pos_sh, posb_v, sa2p_v, vals_v, swb_v,
                   rows_v, rows2_v, sem, sem2):
    sid = lax.axis_index("s")
    de1 = pltpu.make_async_copy(e1_hbm, e1_v, sem)
    de2 = pltpu.make_async_copy(e2_hbm, e2_v, sem2)
    de1.start()
    de2.start()
    lanes = lax.broadcasted_iota(jnp.int32, (L,), 0)
    zero_vec = jnp.zeros((L,), jnp.int32)

    def _shift(v, k):
        sh = v.at[jnp.maximum(lanes - k, 0)].get(mode="promise_in_bounds")
        return v + jnp.where(lanes >= k, sh, 0)

    def _prefix(v):
        for k in (1, 2, 4, 8):
            v = _shift(v, k)
        return v

    def _splat(v, i):
        return v.at[zero_vec + i].get(mode="promise_in_bounds")

    def init_body(i, c):
        stok_v[pl.ds(i * L, L)] = zero_vec
        return c
    lax.fori_loop(0, CAP // L, init_body, 0)
    de1.wait()
    de2.wait()
    dw1 = pltpu.make_async_copy(w1_hbm, w1_v, sem)
    dw2 = pltpu.make_async_copy(w2_hbm, w2_v, sem2)
    dw1.start()
    dw2.start()

    def scan_pass(e_v):
        def body(s2, cv):
            o = s2 * (2 * L)
            ids_a = e_v[pl.ds(o, L)]
            ids_b = e_v[pl.ds(o + L, L)]
            m_a = ids_a == sid
            m_b = ids_b == sid
            pref_a = _prefix(jnp.where(m_a, 1, 0))
            pref_b = _prefix(jnp.where(m_b, 1, 0))
            tot_a = _splat(pref_a, L - 1)
            plsc.store_scatter(stok_v, [cv + pref_a - 1], o + lanes, mask=m_a)
            plsc.store_scatter(stok_v, [cv + tot_a + pref_b - 1],
                               o + L + lanes, mask=m_b)
            return cv + tot_a + _splat(pref_b, L - 1)
        return body

    cv1 = lax.fori_loop(0, T // (2 * L), scan_pass(e1_v), zero_vec)
    cv = lax.fori_loop(0, T // (2 * L), scan_pass(e2_v), cv1)

    dw1.wait()
    dw2.wait()

    tmp_v[...] = jnp.where(lanes == sid, cv, 0)
    pltpu.sync_copy(tmp_v, counts_sh.at[sid])
    plsc.subcore_barrier()
    pltpu.sync_copy(counts_sh, allc_v)
    counts = jnp.zeros((L,), jnp.int32)
    for _e in range(NS):
        counts = counts + allc_v[_e]
    padded = ((counts + (CB - 1)) >> 8) << 8
    pincl = _prefix(padded)
    basev = pincl - padded
    cursor = cv[0]
    c1 = cv1[0]
    base = _splat(basev, sid)[0]

    nch = (cursor + (GC - 1)) >> 5

    def _gd(c, buf, s):
        idx = stok_v.at[pl.ds(pl.multiple_of(c * GC, L), GC)]
        return pltpu.make_async_copy(x_hbm.at[idx], buf, s)

    def _write(c, buf):
        pltpu.sync_copy(
            buf, xg_hbm.at[pl.ds(pl.multiple_of(base + c * GC, GC), GC)])

    def gather_pair(p, _):
        c0 = p * 2
        d0 = _gd(c0, rows_v, sem)
        d1 = _gd(c0 + 1, rows2_v, sem2)
        d0.start()
        d1.start()
        d0.wait()

        @pl.when(c0 < nch)
        def _w0():
            _write(c0, rows_v)
        d1.wait()

        @pl.when(c0 + 1 < nch)
        def _w1():
            _write(c0 + 1, rows2_v)
        return 0
    lax.fori_loop(0, (nch + 1) >> 1, gather_pair, 0)

    def wg_chunk(j, _):
        for q in range(CB // L):
            r0 = pl.multiple_of(j * CB, CB) + q * L
            toks = stok_v[pl.ds(r0, L)]
            wa = plsc.load_gather(w1_v, [toks])
            wb = plsc.load_gather(w2_v, [toks])
            swb_v[pl.ds(q * L, L)] = jnp.where(r0 + lanes >= c1, wb, wa)
        pltpu.sync_copy(
            swb_v, wg_hbm.at[pl.ds(pl.multiple_of(base + j * CB, CB), CB)])
        return 0
    lax.fori_loop(0, (cursor + (CB - 1)) >> 8, wg_chunk, 0)

    def pos_chunk(c, _):
        for q in range(PC // L):
            r0 = pl.multiple_of(c * PC, PC) + q * L
            r = r0 + lanes
            toks = stok_v[pl.ds(r0, L)]
            slot = toks + jnp.where(r >= c1, T, 0)
            slot = jnp.where(r >= cursor, SENT, slot)
            sa2p_v.at[c][pl.ds(q * L, L)] = slot
            vals_v[pl.ds(q * L, L)] = base + r
        d = pltpu.make_async_copy(vals_v, pos_sh.at[sa2p_v.at[c]], sem)
        d.start()
        d.wait()
        return 0
    lax.fori_loop(0, (cursor + (PC - 1)) >> 7, pos_chunk, 0)
    plsc.subcore_barrier()
    psl = POSN // NS
    pltpu.sync_copy(pos_sh.at[pl.ds(pl.multiple_of(sid * psl, psl), psl)],
                    posb_v)
    pltpu.sync_copy(posb_v,
                    pos_hbm.at[pl.ds(pl.multiple_of(sid * psl, psl), psl)])

    @pl.when(sid == EP - 1)
    def _tail():
        bblk = basev >> 8

        def be_slice(s, _):
            bv = s * L + lanes
            acc = jnp.zeros((L,), jnp.int32)
            for _e in range(EP):
                acc = acc + jnp.where(bv >= _splat(bblk, _e), 1, 0)
            tmp_v[...] = acc - 1
            pltpu.sync_copy(tmp_v, be_hbm.at[pl.ds(pl.multiple_of(s * L, L), L)])
            return 0
        lax.fori_loop(0, NB // L, be_slice, 0)


def _dispatch(e1, e2, w1, w2, xs):
    mesh = plsc.VectorSubcoreMesh(core_axis_name="c", subcore_axis_name="s",
                                  num_cores=1)
    f = pl.kernel(
        _dispatch_body,
        mesh=mesh,
        compiler_params=pltpu.CompilerParams(needs_layout_passes=False),
        out_type=[
            jax.ShapeDtypeStruct((PADT, H), jnp.float32),
            jax.ShapeDtypeStruct((PADT,), jnp.float32),
            jax.ShapeDtypeStruct((POSN,), jnp.int32),
            jax.ShapeDtypeStruct((NB,), jnp.int32),
        ],
        scratch_types=[
            pltpu.VMEM((T,), jnp.int32),
            pltpu.VMEM((T,), jnp.int32),
            pltpu.VMEM((T,), jnp.float32),
            pltpu.VMEM((T,), jnp.float32),
            pltpu.VMEM((CAP,), jnp.int32),
            pltpu.VMEM((L,), jnp.int32),
            pltpu.VMEM((NS, L), jnp.int32),
            pltpu.VMEM_SHARED((NS, L), jnp.int32),
            pltpu.VMEM_SHARED((POSN,), jnp.int32),
            pltpu.VMEM((POSN // NS,), jnp.int32),
            pltpu.VMEM((T // PC, PC), jnp.int32),
            pltpu.VMEM((PC,), jnp.int32),
            pltpu.VMEM((CB,), jnp.float32),
            pltpu.VMEM((GC, H), jnp.float32),
            pltpu.VMEM((GC, H), jnp.float32),
            pltpu.SemaphoreType.DMA,
            pltpu.SemaphoreType.DMA,
        ],
    )
    return f(e1, e2, w1, w2, xs)


def _group_body(be_ref, xg_ref, wg_ref, rg_ref, ru_ref, rd_ref, yg_ref):
    xb = xg_ref[...].astype(jnp.bfloat16)
    g = _dotT(xb, rg_ref[0])
    u = _dotT(xb, ru_ref[0])
    h = (_silu(g) * u * wg_ref[...]).astype(jnp.bfloat16)
    yg_ref[...] = _dotT(h, rd_ref[0])


def _grouped(be, xg, wg, rg_pad, ru_pad, rd_pad):
    grid_spec = pltpu.PrefetchScalarGridSpec(
        num_scalar_prefetch=1,
        grid=(NB,),
        in_specs=[
            pl.BlockSpec((CB, H), lambda j, be: (j, 0)),
            pl.BlockSpec((CB, 1), lambda j, be: (j, 0)),
            pl.BlockSpec((1, I, H), lambda j, be: (be[j], 0, 0)),
            pl.BlockSpec((1, I, H), lambda j, be: (be[j], 0, 0)),
            pl.BlockSpec((1, H, I), lambda j, be: (be[j], 0, 0)),
        ],
        out_specs=pl.BlockSpec((CB, H), lambda j, be: (j, 0)),
    )
    return pl.pallas_call(
        _group_body,
        grid_spec=grid_spec,
        out_shape=jax.ShapeDtypeStruct((PADT, H), jnp.float32),
    )(be, xg, wg.reshape(PADT, 1), rg_pad, ru_pad, rd_pad)


def _gcombine_body(yg_hbm, pos_hbm, y0_hbm, y1_hbm,
                   pos_v, rows_v, rows2_v, sem, sem2):
    wid = lax.axis_index("c") * NS + lax.axis_index("s")
    t0 = pl.multiple_of(wid * (T // 32), T // 32)
    pltpu.sync_copy(pos_hbm.at[pl.ds(t0, T // 32)], pos_v.at[0])
    pltpu.sync_copy(pos_hbm.at[pl.ds(T + t0, T // 32)], pos_v.at[1])
    nc = (T // 32) // SC_C

    def _gd(k, c, buf, s):
        idx = pos_v.at[k, pl.ds(c * SC_C, SC_C)]
        return pltpu.make_async_copy(yg_hbm.at[idx], buf, s)

    def _wr(dst, c, buf):
        pltpu.sync_copy(
            buf, dst.at[pl.ds(pl.multiple_of(t0 + c * SC_C, SC_C), SC_C)])

    for k, dst in ((0, y0_hbm), (1, y1_hbm)):
        for c in range(0, nc, 2):
            d0 = _gd(k, c, rows_v, sem)
            d1 = _gd(k, c + 1, rows2_v, sem2)
            d0.start()
            d1.start()
            d0.wait()
            _wr(dst, c, rows_v)
            d1.wait()
            _wr(dst, c + 1, rows2_v)


def _gcombine(yg, pos):
    mesh = plsc.VectorSubcoreMesh(core_axis_name="c", subcore_axis_name="s")
    f = pl.kernel(
        _gcombine_body,
        mesh=mesh,
        compiler_params=pltpu.CompilerParams(needs_layout_passes=False),
        out_type=[
            jax.ShapeDtypeStruct((T, H), jnp.float32),
            jax.ShapeDtypeStruct((T, H), jnp.float32),
        ],
        scratch_types=[
            pltpu.VMEM((2, T // 32), jnp.int32),
            pltpu.VMEM((SC_C, H), jnp.float32),
            pltpu.VMEM((SC_C, H), jnp.float32),
            pltpu.SemaphoreType.DMA,
            pltpu.SemaphoreType.DMA,
        ],
    )
    return f(yg, pos)


def _shared_body(x_ref, sg_ref, su_ref, sd_ref, out_ref):
    x = x_ref[...]
    g = _dotT(x, sg_ref[...])
    u = _dotT(x, su_ref[...])
    out_ref[...] = _dotT(_silu(g) * u, sd_ref[...])


def _shared(xs, sg, su, sd):
    return pl.pallas_call(
        _shared_body,
        grid=(T // TB,),
        in_specs=[
            pl.BlockSpec((TB, H), lambda i: (i, 0)),
            pl.BlockSpec((I, H), lambda i: (0, 0)),
            pl.BlockSpec((I, H), lambda i: (0, 0)),
            pl.BlockSpec((H, I), lambda i: (0, 0)),
        ],
        out_specs=pl.BlockSpec((TB, H), lambda i: (i, 0)),
        out_shape=jax.ShapeDtypeStruct((T, H), jnp.float32),
    )(xs, sg, su, sd)


def _combine_body(sh_ref, y0_ref, y1_ref, out_ref):
    out_ref[...] = sh_ref[...] + y0_ref[...] + y1_ref[...]


def _combine(sh, y0, y1):
    return pl.pallas_call(
        _combine_body,
        grid=(T // TB,),
        in_specs=[
            pl.BlockSpec((TB, H), lambda i: (i, 0)),
            pl.BlockSpec((TB, H), lambda i: (i, 0)),
            pl.BlockSpec((TB, H), lambda i: (i, 0)),
        ],
        out_specs=pl.BlockSpec((TB, H), lambda i: (i, 0)),
        out_shape=jax.ShapeDtypeStruct((T, H), jnp.float32),
    )(sh, y0, y1)


def kernel(x, sg, su, sd, rg, ru, rd, Wr, rb):
    orig_shape = x.shape
    xs = x.reshape(-1, H)
    e1, e2, w1, w2 = _router(xs, Wr, rb)
    xg, wg, pos, be = _dispatch(e1.reshape(-1), e2.reshape(-1),
                                w1.reshape(-1), w2.reshape(-1), xs)
    zpad = jnp.zeros((1,) + rg.shape[1:], jnp.bfloat16)
    rg_pad = jnp.concatenate([rg.astype(jnp.bfloat16), zpad], axis=0)
    ru_pad = jnp.concatenate([ru.astype(jnp.bfloat16), zpad], axis=0)
    rd_pad = jnp.concatenate(
        [rd.astype(jnp.bfloat16),
         jnp.zeros((1,) + rd.shape[1:], jnp.bfloat16)], axis=0)
    sh = _shared(xs, sg, su, sd)
    yg = _grouped(be, xg, wg, rg_pad, ru_pad, rd_pad)
    y0, y1 = _gcombine(yg, pos)
    out = _combine(sh, y0, y1)
    return out.reshape(orig_shape)
